# baseline (speedup 1.0000x reference)
import jax
import jax.numpy as jnp
from jax.experimental import pallas as pl
from jax.experimental.pallas import tpu as pltpu

_N = 10000
_XD = 128
_HD = 64
_ZD = 16
_HZ = _HD + _ZD
_YD = 128

_TP = 2000
_TR = 400
_NR = _N // _TR

_HIGH = jax.lax.Precision.HIGHEST


def _prep_body(x_ref, w1_ref, b1_ref, z_ref, w2_ref, g1_ref):
    h = jnp.dot(x_ref[...], w1_ref[...], precision=_HIGH,
                preferred_element_type=jnp.float32) + b1_ref[...]
    h = jnp.maximum(h, 0.0)
    z = z_ref[...]
    zsq = jnp.sum(z * z)
    inv = 1.0 / (jnp.sqrt(jnp.sum(h * h, axis=1, keepdims=True) + zsq) + 1e-6)
    w2 = w2_ref[...]
    hw = jnp.dot(h, w2[:_HD, :], precision=_HIGH,
                 preferred_element_type=jnp.float32)
    zw = jnp.dot(z, w2[_HD:, :], precision=_HIGH,
                 preferred_element_type=jnp.float32)
    g1_ref[...] = ((hw + zw) * inv).astype(jnp.bfloat16)


def _pass1_body(adj_ref, g1_ref, b2_ref, wy_ref, out_ref):
    a = adj_ref[...].astype(jnp.bfloat16)
    acc = jnp.dot(a, g1_ref[...], preferred_element_type=jnp.float32)
    h2 = jnp.maximum(acc + b2_ref[...], 0.0)
    g2 = jnp.dot(h2, wy_ref[...], precision=_HIGH,
                 preferred_element_type=jnp.float32)
    out_ref[...] = g2.astype(jnp.bfloat16)


def _pass2_body(adj_ref, g2_ref, by_ref, out_ref):
    a = adj_ref[...].astype(jnp.bfloat16)
    acc = jnp.dot(a, g2_ref[...], preferred_element_type=jnp.float32)
    out_ref[...] = acc + by_ref[...]


def kernel(x, adj, z, W1, b1, W2, b2, Wy, by):
    z2 = z.reshape(1, _ZD)
    b1_2 = b1.reshape(1, _HD)
    b2_2 = b2.reshape(1, _HZ)
    by_2 = by.reshape(1, _YD)

    g1 = pl.pallas_call(
        _prep_body,
        grid=(_N // _TP,),
        in_specs=[
            pl.BlockSpec((_TP, _XD), lambda i: (i, 0)),
            pl.BlockSpec((_XD, _HD), lambda i: (0, 0)),
            pl.BlockSpec((1, _HD), lambda i: (0, 0)),
            pl.BlockSpec((1, _ZD), lambda i: (0, 0)),
            pl.BlockSpec((_HZ, _HZ), lambda i: (0, 0)),
        ],
        out_specs=pl.BlockSpec((_TP, _HZ), lambda i: (i, 0)),
        out_shape=jax.ShapeDtypeStruct((_N, _HZ), jnp.bfloat16),
    )(x, W1, b1_2, z2, W2)

    adj_spec = pl.BlockSpec((_TR, _N), lambda i: (i, 0))
    cparams = pltpu.CompilerParams(dimension_semantics=("arbitrary",))

    g2 = pl.pallas_call(
        _pass1_body,
        grid=(_NR,),
        in_specs=[
            adj_spec,
            pl.BlockSpec((_N, _HZ), lambda i: (0, 0)),
            pl.BlockSpec((1, _HZ), lambda i: (0, 0)),
            pl.BlockSpec((_HZ, _YD), lambda i: (0, 0)),
        ],
        out_specs=pl.BlockSpec((_TR, _YD), lambda i: (i, 0)),
        out_shape=jax.ShapeDtypeStruct((_N, _YD), jnp.bfloat16),
        compiler_params=cparams,
    )(adj, g1, b2_2, Wy)

    y = pl.pallas_call(
        _pass2_body,
        grid=(_NR,),
        in_specs=[
            adj_spec,
            pl.BlockSpec((_N, _YD), lambda i: (0, 0)),
            pl.BlockSpec((1, _YD), lambda i: (0, 0)),
        ],
        out_specs=pl.BlockSpec((_TR, _YD), lambda i: (i, 0)),
        out_shape=jax.ShapeDtypeStruct((_N, _YD), jnp.float32),
        compiler_params=cparams,
    )(adj, g2, by_2)

    return y
